# baseline (speedup 1.0000x reference)
import functools

import jax
import jax.numpy as jnp
from jax import lax
from jax.experimental import pallas as pl
from jax.experimental.pallas import tpu as pltpu


def _halo(w):
    return ((w + 1 + 7) // 8) * 8


def _conv3x3_cm_in(hsc, isc, w_ref, b_ref, x, Wl, HWl, P, pad_val):
    cin = x.shape[0]
    fill = jnp.full((cin, P), pad_val, dtype=hsc.dtype)
    hsc[:, 0:P] = fill
    hsc[:, P:P + HWl] = x.astype(hsc.dtype)
    hsc[:, P + HWl:P + HWl + P] = fill

    jcol = lax.broadcasted_iota(jnp.int32, (1, HWl), 1) % Wl
    left_ok = jcol > 0
    right_ok = jcol < Wl - 1

    for di in range(3):
        for dj in range(3):
            t = di * 3 + dj
            off = P + (di - 1) * Wl + (dj - 1)
            v = hsc[:, pl.ds(off, HWl)]
            if dj == 0:
                v = jnp.where(left_ok, v, pad_val)
            elif dj == 2:
                v = jnp.where(right_ok, v, pad_val)
            isc[t * cin:(t + 1) * cin, :] = v
    return lax.dot_general(
        isc[...], w_ref[...], (((0,), (0,)), ((), ())),
        preferred_element_type=jnp.float32) + b_ref[...]


def _conv3x3(hsc, isc, w_ref, b_ref, x, Wl, HWl, P, pad_val):
    cin = x.shape[1]
    fill = jnp.full((P, cin), pad_val, dtype=hsc.dtype)
    hsc[0:P, :] = fill
    hsc[P:P + HWl, :] = x.astype(hsc.dtype)
    hsc[P + HWl:P + HWl + P, :] = fill

    jcol = lax.broadcasted_iota(jnp.int32, (HWl, 1), 0) % Wl
    left_ok = jcol > 0
    right_ok = jcol < Wl - 1

    for di in range(3):
        for dj in range(3):
            t = di * 3 + dj
            off = P + (di - 1) * Wl + (dj - 1)
            v = hsc[pl.ds(off, HWl), :]
            if dj == 0:
                v = jnp.where(left_ok, v, pad_val)
            elif dj == 2:
                v = jnp.where(right_ok, v, pad_val)
            isc[:, t * cin:(t + 1) * cin] = v
    return jnp.dot(isc[...], w_ref[...],
                   preferred_element_type=jnp.float32) + b_ref[...]


def _point_net_kernel(
    x_ref, w1_ref, b1_ref, w2_ref, b2_ref, w3_ref, b3_ref, w4_ref, b4_ref,
    wh_ref, bh_ref,
    score_ref, coord_ref, desc_ref,
    hs1, ic1, hs2, ic2, ds2, hs3, ic3, hs4, ic4, ds4, hsh, ich,
    *, H, W, n_valid,
):
    HW = H * W
    H2, W2 = H // 2, W // 2
    HW2 = H2 * W2
    hc, wc = H // 4, W // 4
    HWc = hc * wc
    P1, P2, P3 = _halo(W), _halo(W2), _halo(wc)
    c2 = w2_ref.shape[1]
    c4 = w4_ref.shape[1]

    x0 = x_ref[0]
    a1 = jnp.maximum(
        _conv3x3_cm_in(hs1, ic1, w1_ref, b1_ref, x0, W, HW, P1, 0.5), 0.0)
    a2 = jnp.maximum(_conv3x3(hs2, ic2, w2_ref, b2_ref, a1, W, HW, P1, 0.0), 0.0)

    ds2[...] = a2.reshape(H2, 2 * W, c2).astype(ds2.dtype)
    a2d = ds2[:, pl.ds(0, W2, 2), :].reshape(HW2, c2)

    a3 = jnp.maximum(_conv3x3(hs3, ic3, w3_ref, b3_ref, a2d, W2, HW2, P2, 0.0), 0.0)
    a4 = jnp.maximum(_conv3x3(hs4, ic4, w4_ref, b4_ref, a3, W2, HW2, P2, 0.0), 0.0)

    ds4[...] = a4.reshape(H2 // 2, 2 * W2, c4).astype(ds4.dtype)
    feat = ds4[:, pl.ds(0, wc, 2), :].reshape(HWc, c4)

    h = _conv3x3(hsh, ich, wh_ref, bh_ref, feat, wc, HWc, P3, 0.0)

    col = lax.broadcasted_iota(jnp.int32, h.shape, 1)
    cell = H // hc
    step = (cell - 1) / 2.0
    k = lax.broadcasted_iota(jnp.int32, (HWc, 1), 0)
    bx = (k % wc).astype(jnp.float32) * cell + step
    by = (k // wc).astype(jnp.float32) * cell + step
    base = jnp.where(col == 1, bx, by)
    cmax = jnp.where(col == 1, float(W - 1), float(H - 1))
    coord = jnp.clip(base + jnp.tanh(h) * step, 0.0, cmax)
    score = jax.nn.sigmoid(h)
    out = jnp.where(col == 0, score, jnp.where(col <= 2, coord, h))
    out_t = out.T
    score_ref[0] = out_t[0:1, :]
    coord_ref[0] = out_t[1:3, :]
    desc_ref[0] = out_t[3:n_valid, :]


def kernel(img, w1, b1, w2, b2, w3, b3, w4, b4, ws, bs, wc, bc, wd, bd):
    B, C, H, W = img.shape
    hc, wcell = H // 4, W // 4
    HW, HW2, HWc = H * W, (H // 2) * (W // 2), hc * wcell
    c1 = w1.shape[1]
    c2 = w2.shape[1]
    c3 = w3.shape[1]
    c4 = w4.shape[1]
    c_desc = wd.shape[1]
    n_valid = 3 + c_desc
    NH = 128

    x = img.reshape(B, C, HW)

    inv_std = 1.0 / 0.225
    w1f = w1 * inv_std
    b1f = b1 - (0.5 * inv_std) * jnp.sum(w1, axis=0, keepdims=True)

    wh = jnp.concatenate([ws, wc, wd], axis=1)
    bh = jnp.concatenate([bs, bc, bd], axis=1)
    wh = jnp.pad(wh, ((0, 0), (0, NH - n_valid)))
    bh = jnp.pad(bh, ((0, 0), (0, NH - n_valid)))

    P1, P2, P3 = _halo(W), _halo(W // 2), _halo(wcell)
    full = lambda a: pl.BlockSpec(a.shape, lambda b: (0, 0))
    f32 = jnp.float32
    bf16 = jnp.bfloat16
    w2c, w3c, w4c, whc = (a.astype(bf16) for a in (w2, w3, w4, wh))

    score, coord, desc = pl.pallas_call(
        functools.partial(_point_net_kernel, H=H, W=W, n_valid=n_valid),
        out_shape=(
            jax.ShapeDtypeStruct((B, 1, HWc), f32),
            jax.ShapeDtypeStruct((B, 2, HWc), f32),
            jax.ShapeDtypeStruct((B, c_desc, HWc), f32),
        ),
        grid=(B,),
        in_specs=[
            pl.BlockSpec((1, C, HW), lambda b: (b, 0, 0)),
            full(w1f), full(b1f),
            full(w2), full(b2),
            full(w3), full(b3),
            full(w4), full(b4),
            full(wh), full(bh),
        ],
        out_specs=(
            pl.BlockSpec((1, 1, HWc), lambda b: (b, 0, 0)),
            pl.BlockSpec((1, 2, HWc), lambda b: (b, 0, 0)),
            pl.BlockSpec((1, c_desc, HWc), lambda b: (b, 0, 0)),
        ),
        scratch_shapes=[
            pltpu.VMEM((C, HW + 2 * P1), f32),
            pltpu.VMEM((9 * C, HW), f32),
            pltpu.VMEM((HW + 2 * P1, c1), bf16),
            pltpu.VMEM((HW, 9 * c1), bf16),
            pltpu.VMEM((H // 2, 2 * W, c2), f32),
            pltpu.VMEM((HW2 + 2 * P2, c2), bf16),
            pltpu.VMEM((HW2, 9 * c2), bf16),
            pltpu.VMEM((HW2 + 2 * P2, c3), bf16),
            pltpu.VMEM((HW2, 9 * c3), bf16),
            pltpu.VMEM((H // 4, 2 * (W // 2), c4), f32),
            pltpu.VMEM((HWc + 2 * P3, c4), bf16),
            pltpu.VMEM((HWc, 9 * c4), bf16),
        ],
        compiler_params=pltpu.CompilerParams(
            dimension_semantics=("parallel",),
            vmem_limit_bytes=64 * 1024 * 1024,
        ),
    )(x, w1f, b1f, w2c, b2, w3c, b3, w4c, b4, whc, bh)

    return (score.reshape(B, 1, hc, wcell),
            coord.reshape(B, 2, hc, wcell),
            desc.reshape(B, c_desc, hc, wcell))


# speedup vs baseline: 2.0510x; 2.0510x over previous
import functools

import jax
import jax.numpy as jnp
from jax import lax
from jax.experimental import pallas as pl
from jax.experimental.pallas import tpu as pltpu


def _halo(w):
    return ((w + 1 + 7) // 8) * 8


def _stage_cm(hsc, isc, x, Wl, HWl, P, pad_val):
    cin = x.shape[0]
    fill = jnp.full((cin, P), pad_val, dtype=jnp.float32)
    hsc[:, 0:P] = fill
    hsc[:, P:P + HWl] = x
    hsc[:, P + HWl:P + HWl + P] = fill

    jcol = lax.broadcasted_iota(jnp.int32, (1, HWl), 1) % Wl
    left_ok = jcol > 0
    right_ok = jcol < Wl - 1

    for di in range(3):
        for dj in range(3):
            t = di * 3 + dj
            off = P + (di - 1) * Wl + (dj - 1)
            v = hsc[:, pl.ds(off, HWl)]
            if dj == 0:
                v = jnp.where(left_ok, v, pad_val)
            elif dj == 2:
                v = jnp.where(right_ok, v, pad_val)
            isc[t * cin:(t + 1) * cin, :] = v


def _conv3x3_cm(hsc, isc, wT_ref, bT_ref, x, Wl, HWl, P, pad_val):
    _stage_cm(hsc, isc, x, Wl, HWl, P, pad_val)
    return jnp.dot(wT_ref[...], isc[...],
                   preferred_element_type=jnp.float32) + bT_ref[...]


def _conv3x3_cm_rm(hsc, isc, w_ref, b_ref, x, Wl, HWl, P, pad_val):
    _stage_cm(hsc, isc, x, Wl, HWl, P, pad_val)
    return lax.dot_general(
        isc[...], w_ref[...], (((0,), (0,)), ((), ())),
        preferred_element_type=jnp.float32) + b_ref[...]


def _point_net_kernel(
    x_ref, w1T_ref, b1T_ref, w2_ref, b2_ref, w3T_ref, b3T_ref,
    w4_ref, b4_ref, whT_ref, bhT_ref,
    score_ref, coord_ref, desc_ref,
    hs1, ic1, hs2, ic2, ds2, hs3, ic3, hs4, ic4, ds4, hsh, ich,
    *, H, W, n_valid, nb,
):
    HW = H * W
    H2, W2 = H // 2, W // 2
    HW2 = H2 * W2
    hc, wc = H // 4, W // 4
    HWc = hc * wc
    P1, P2, P3 = _halo(W), _halo(W2), _halo(wc)
    c2 = w2_ref.shape[1]
    c4 = w4_ref.shape[1]

    for img in range(nb):
      x0 = x_ref[img]
      a1 = jnp.maximum(
          _conv3x3_cm(hs1, ic1, w1T_ref, b1T_ref, x0, W, HW, P1, 0.5), 0.0)
      a2 = jnp.maximum(
          _conv3x3_cm_rm(hs2, ic2, w2_ref, b2_ref, a1, W, HW, P1, 0.0), 0.0)

      ds2[...] = a2.reshape(H2, 2 * W, c2)
      a2d = ds2[:, pl.ds(0, W2, 2), :].reshape(HW2, c2).T

      a3 = jnp.maximum(
          _conv3x3_cm(hs3, ic3, w3T_ref, b3T_ref, a2d, W2, HW2, P2, 0.0), 0.0)
      a4 = jnp.maximum(
          _conv3x3_cm_rm(hs4, ic4, w4_ref, b4_ref, a3, W2, HW2, P2, 0.0), 0.0)

      ds4[...] = a4.reshape(H2 // 2, 2 * W2, c4)
      feat = ds4[:, pl.ds(0, wc, 2), :].reshape(HWc, c4).T

      h = _conv3x3_cm(hsh, ich, whT_ref, bhT_ref, feat, wc, HWc, P3, 0.0)

      row = lax.broadcasted_iota(jnp.int32, h.shape, 0)
      cell = H // hc
      step = (cell - 1) / 2.0
      k = lax.broadcasted_iota(jnp.int32, (1, HWc), 1)
      bx = (k % wc).astype(jnp.float32) * cell + step
      by = (k // wc).astype(jnp.float32) * cell + step
      base = jnp.where(row == 1, bx, by)
      cmax = jnp.where(row == 1, float(W - 1), float(H - 1))
      coord = jnp.clip(base + jnp.tanh(h) * step, 0.0, cmax)
      score = jax.nn.sigmoid(h)
      out = jnp.where(row == 0, score, jnp.where(row <= 2, coord, h))
      score_ref[img] = out[0:1, :]
      coord_ref[img] = out[1:3, :]
      desc_ref[img] = out[3:n_valid, :]


def kernel(img, w1, b1, w2, b2, w3, b3, w4, b4, ws, bs, wc, bc, wd, bd):
    B, C, H, W = img.shape
    hc, wcell = H // 4, W // 4
    HW, HW2, HWc = H * W, (H // 2) * (W // 2), hc * wcell
    c1 = w1.shape[1]
    c2 = w2.shape[1]
    c3 = w3.shape[1]
    c4 = w4.shape[1]
    c_desc = wd.shape[1]
    n_valid = 3 + c_desc
    NH = 128

    x = img.reshape(B, C, HW)

    inv_std = 1.0 / 0.225
    w1f = w1 * inv_std
    b1f = b1 - (0.5 * inv_std) * jnp.sum(w1, axis=0, keepdims=True)

    wh = jnp.concatenate([ws, wc, wd], axis=1)
    bh = jnp.concatenate([bs, bc, bd], axis=1)
    wh = jnp.pad(wh, ((0, 0), (0, NH - n_valid)))
    bh = jnp.pad(bh, ((0, 0), (0, NH - n_valid)))

    nb = 4 if B % 4 == 0 else 1
    P1, P2, P3 = _halo(W), _halo(W // 2), _halo(wcell)
    full = lambda a: pl.BlockSpec(a.shape, lambda b: tuple(0 for _ in a.shape))
    f32 = jnp.float32

    score, coord, desc = pl.pallas_call(
        functools.partial(_point_net_kernel, H=H, W=W, n_valid=n_valid, nb=nb),
        out_shape=(
            jax.ShapeDtypeStruct((B, 1, HWc), f32),
            jax.ShapeDtypeStruct((B, 2, HWc), f32),
            jax.ShapeDtypeStruct((B, c_desc, HWc), f32),
        ),
        grid=(B // nb,),
        in_specs=[
            pl.BlockSpec((nb, C, HW), lambda b: (b, 0, 0)),
            full(w1f.T), full(b1f.T),
            full(w2), full(b2),
            full(w3.T), full(b3.T),
            full(w4), full(b4),
            full(wh.T), full(bh.T),
        ],
        out_specs=(
            pl.BlockSpec((nb, 1, HWc), lambda b: (b, 0, 0)),
            pl.BlockSpec((nb, 2, HWc), lambda b: (b, 0, 0)),
            pl.BlockSpec((nb, c_desc, HWc), lambda b: (b, 0, 0)),
        ),
        scratch_shapes=[
            pltpu.VMEM((C, HW + 2 * P1), f32),
            pltpu.VMEM((9 * C, HW), f32),
            pltpu.VMEM((c1, HW + 2 * P1), f32),
            pltpu.VMEM((9 * c1, HW), f32),
            pltpu.VMEM((H // 2, 2 * W, c2), f32),
            pltpu.VMEM((c2, HW2 + 2 * P2), f32),
            pltpu.VMEM((9 * c2, HW2), f32),
            pltpu.VMEM((c3, HW2 + 2 * P2), f32),
            pltpu.VMEM((9 * c3, HW2), f32),
            pltpu.VMEM((H // 4, 2 * (W // 2), c4), f32),
            pltpu.VMEM((c4, HWc + 2 * P3), f32),
            pltpu.VMEM((9 * c4, HWc), f32),
        ],
        compiler_params=pltpu.CompilerParams(
            dimension_semantics=("parallel",),
            vmem_limit_bytes=64 * 1024 * 1024,
        ),
    )(x, w1f.T, b1f.T, w2, b2, w3.T, b3.T, w4, b4, wh.T, bh.T)

    return (score.reshape(B, 1, hc, wcell),
            coord.reshape(B, 2, hc, wcell),
            desc.reshape(B, c_desc, hc, wcell))


# speedup vs baseline: 2.0535x; 1.0012x over previous
import functools

import jax
import jax.numpy as jnp
from jax import lax
from jax.experimental import pallas as pl
from jax.experimental.pallas import tpu as pltpu


def _halo(w):
    return ((w + 1 + 7) // 8) * 8


def _stage_cm(hsc, isc, x, Wl, HWl, P, pad_val):
    cin = x.shape[0]
    fill = jnp.full((cin, P), pad_val, dtype=jnp.float32)
    hsc[:, 0:P] = fill
    hsc[:, P:P + HWl] = x
    hsc[:, P + HWl:P + HWl + P] = fill

    jcol = lax.broadcasted_iota(jnp.int32, (1, HWl), 1) % Wl
    left_ok = jcol > 0
    right_ok = jcol < Wl - 1

    for di in range(3):
        for dj in range(3):
            t = di * 3 + dj
            off = P + (di - 1) * Wl + (dj - 1)
            v = hsc[:, pl.ds(off, HWl)]
            if dj == 0:
                v = jnp.where(left_ok, v, pad_val)
            elif dj == 2:
                v = jnp.where(right_ok, v, pad_val)
            isc[t * cin:(t + 1) * cin, :] = v


def _conv3x3_cm(hsc, isc, wT_ref, bT_ref, x, Wl, HWl, P, pad_val):
    _stage_cm(hsc, isc, x, Wl, HWl, P, pad_val)
    return jnp.dot(wT_ref[...], isc[...],
                   preferred_element_type=jnp.float32) + bT_ref[...]


def _conv3x3_cm_rm(hsc, isc, w_ref, b_ref, x, Wl, HWl, P, pad_val):
    _stage_cm(hsc, isc, x, Wl, HWl, P, pad_val)
    return lax.dot_general(
        isc[...], w_ref[...], (((0,), (0,)), ((), ())),
        preferred_element_type=jnp.float32) + b_ref[...]


def _point_net_kernel(
    x_ref, w1T_ref, b1T_ref, w2_ref, b2_ref, w3T_ref, b3T_ref,
    w4_ref, b4_ref, whT_ref, bhT_ref,
    score_ref, coord_ref, desc_ref,
    *scr,
    H, W, n_valid, nb,
):
    HW = H * W
    H2, W2 = H // 2, W // 2
    HW2 = H2 * W2
    hc, wc = H // 4, W // 4
    HWc = hc * wc
    P1, P2, P3 = _halo(W), _halo(W2), _halo(wc)
    c2 = w2_ref.shape[1]
    c4 = w4_ref.shape[1]

    for img in range(nb):
      (hs1i, ic1i, hs2i, ic2i, ds2i, hs3i, ic3i, hs4i, ic4i, ds4i, hshi,
       ichi) = scr[12 * (img % 2):12 * (img % 2) + 12]
      x0 = x_ref[img]
      a1 = jnp.maximum(
          _conv3x3_cm(hs1i, ic1i, w1T_ref, b1T_ref, x0, W, HW, P1, 0.5), 0.0)
      a2 = jnp.maximum(
          _conv3x3_cm_rm(hs2i, ic2i, w2_ref, b2_ref, a1, W, HW, P1, 0.0), 0.0)

      ds2i[...] = a2.reshape(H2, 2 * W, c2)
      a2d = ds2i[:, pl.ds(0, W2, 2), :].reshape(HW2, c2).T

      a3 = jnp.maximum(
          _conv3x3_cm(hs3i, ic3i, w3T_ref, b3T_ref, a2d, W2, HW2, P2, 0.0), 0.0)
      a4 = jnp.maximum(
          _conv3x3_cm_rm(hs4i, ic4i, w4_ref, b4_ref, a3, W2, HW2, P2, 0.0), 0.0)

      ds4i[...] = a4.reshape(H2 // 2, 2 * W2, c4)
      feat = ds4i[:, pl.ds(0, wc, 2), :].reshape(HWc, c4).T

      h = _conv3x3_cm(hshi, ichi, whT_ref, bhT_ref, feat, wc, HWc, P3, 0.0)

      row = lax.broadcasted_iota(jnp.int32, h.shape, 0)
      cell = H // hc
      step = (cell - 1) / 2.0
      k = lax.broadcasted_iota(jnp.int32, (1, HWc), 1)
      bx = (k % wc).astype(jnp.float32) * cell + step
      by = (k // wc).astype(jnp.float32) * cell + step
      base = jnp.where(row == 1, bx, by)
      cmax = jnp.where(row == 1, float(W - 1), float(H - 1))
      coord = jnp.clip(base + jnp.tanh(h) * step, 0.0, cmax)
      score = jax.nn.sigmoid(h)
      out = jnp.where(row == 0, score, jnp.where(row <= 2, coord, h))
      score_ref[img] = out[0:1, :]
      coord_ref[img] = out[1:3, :]
      desc_ref[img] = out[3:n_valid, :]


def kernel(img, w1, b1, w2, b2, w3, b3, w4, b4, ws, bs, wc, bc, wd, bd):
    B, C, H, W = img.shape
    hc, wcell = H // 4, W // 4
    HW, HW2, HWc = H * W, (H // 2) * (W // 2), hc * wcell
    c1 = w1.shape[1]
    c2 = w2.shape[1]
    c3 = w3.shape[1]
    c4 = w4.shape[1]
    c_desc = wd.shape[1]
    n_valid = 3 + c_desc
    NH = 128

    x = img.reshape(B, C, HW)

    inv_std = 1.0 / 0.225
    w1f = w1 * inv_std
    b1f = b1 - (0.5 * inv_std) * jnp.sum(w1, axis=0, keepdims=True)

    wh = jnp.concatenate([ws, wc, wd], axis=1)
    bh = jnp.concatenate([bs, bc, bd], axis=1)
    wh = jnp.pad(wh, ((0, 0), (0, NH - n_valid)))
    bh = jnp.pad(bh, ((0, 0), (0, NH - n_valid)))

    nb = 4 if B % 4 == 0 else 1
    P1, P2, P3 = _halo(W), _halo(W // 2), _halo(wcell)
    full = lambda a: pl.BlockSpec(a.shape, lambda b: tuple(0 for _ in a.shape))
    f32 = jnp.float32

    score, coord, desc = pl.pallas_call(
        functools.partial(_point_net_kernel, H=H, W=W, n_valid=n_valid, nb=nb),
        out_shape=(
            jax.ShapeDtypeStruct((B, 1, HWc), f32),
            jax.ShapeDtypeStruct((B, 2, HWc), f32),
            jax.ShapeDtypeStruct((B, c_desc, HWc), f32),
        ),
        grid=(B // nb,),
        in_specs=[
            pl.BlockSpec((nb, C, HW), lambda b: (b, 0, 0)),
            full(w1f.T), full(b1f.T),
            full(w2), full(b2),
            full(w3.T), full(b3.T),
            full(w4), full(b4),
            full(wh.T), full(bh.T),
        ],
        out_specs=(
            pl.BlockSpec((nb, 1, HWc), lambda b: (b, 0, 0)),
            pl.BlockSpec((nb, 2, HWc), lambda b: (b, 0, 0)),
            pl.BlockSpec((nb, c_desc, HWc), lambda b: (b, 0, 0)),
        ),
        scratch_shapes=[
            pltpu.VMEM(s, dt)
            for _ in range(2)
            for s, dt in [
                ((C, HW + 2 * P1), f32), ((9 * C, HW), f32),
                ((c1, HW + 2 * P1), f32), ((9 * c1, HW), f32),
                (((H // 2, 2 * W, c2)), f32),
                ((c2, HW2 + 2 * P2), f32), ((9 * c2, HW2), f32),
                ((c3, HW2 + 2 * P2), f32), ((9 * c3, HW2), f32),
                (((H // 4, 2 * (W // 2), c4)), f32),
                ((c4, HWc + 2 * P3), f32), ((9 * c4, HWc), f32),
            ]
        ],
        compiler_params=pltpu.CompilerParams(
            dimension_semantics=("parallel",),
            vmem_limit_bytes=64 * 1024 * 1024,
        ),
    )(x, w1f.T, b1f.T, w2, b2, w3.T, b3.T, w4, b4, wh.T, bh.T)

    return (score.reshape(B, 1, hc, wcell),
            coord.reshape(B, 2, hc, wcell),
            desc.reshape(B, c_desc, hc, wcell))
